# baseline (speedup 1.0000x reference)
import functools

import jax
import jax.numpy as jnp
from jax import lax
from jax.experimental import pallas as pl
from jax.experimental.pallas import tpu as pltpu
from jax.experimental.pallas import tpu_sc as plsc

N_PTS = 1048576
NB = 128
RT = NB * NB * NB * 4 // 16

NC = 2
NS = 16
NW = NC * NS

BLK = 4096
PER_W = N_PTS // NW
NBLK = PER_W // BLK
CHUNK = 2048
NWAVE = BLK * 4 // CHUNK

_mesh = plsc.VectorSubcoreMesh(core_axis_name="c", subcore_axis_name="s")


@functools.partial(
    pl.kernel,
    mesh=_mesh,
    compiler_params=pltpu.CompilerParams(
        needs_layout_passes=False, use_tc_tiling_on_sc=False),
    out_type=[
        jax.ShapeDtypeStruct((N_PTS * 3,), jnp.float32),
        jax.ShapeDtypeStruct((N_PTS,), jnp.float32),
    ],
    scratch_types=[
        pltpu.VMEM((BLK,), jnp.float32),
        pltpu.VMEM((BLK,), jnp.float32),
        pltpu.VMEM((BLK,), jnp.float32),
        pltpu.VMEM((BLK + 16,), jnp.int32),
        pltpu.VMEM((BLK + 16,), jnp.int32),
        pltpu.VMEM((BLK + 16,), jnp.int32),
        pltpu.VMEM((BLK * 4,), jnp.int32),
        pltpu.VMEM((CHUNK, 16), jnp.float32),
        pltpu.VMEM((BLK * 3,), jnp.float32),
        pltpu.VMEM((BLK,), jnp.float32),
        pltpu.SemaphoreType.DMA,
    ],
)
def _voxel_fwd(table_hbm, x_hbm, y_hbm, z_hbm, rgb_hbm, dens_hbm,
               x_v, y_v, z_v, q_v, col_v, pos_v, idx_v, rows_v,
               rgb_v, dens_v, sem):
    wid = lax.axis_index("s") * NC + lax.axis_index("c")
    iota = lax.iota(jnp.int32, 16)
    half = jnp.full((16,), 0.5, jnp.float32)
    zerov = jnp.zeros((16,), jnp.float32)
    zeroi = jnp.zeros((16,), jnp.int32)

    def init_idx(t, c):
        idx_v[pl.ds(t * 16, 16)] = zeroi
        return c
    lax.fori_loop(0, BLK * 4 // 16, init_idx, 0)

    def init_q(t, c):
        q_v[pl.ds(t * 16, 16)] = zeroi
        return c
    lax.fori_loop(0, BLK // 16 + 1, init_q, 0)

    def block_body(b, carry):
        base = (wid * NBLK + b) * BLK
        pltpu.sync_copy(x_hbm.at[pl.ds(base, BLK)], x_v)
        pltpu.sync_copy(y_hbm.at[pl.ds(base, BLK)], y_v)
        pltpu.sync_copy(z_hbm.at[pl.ds(base, BLK)], z_v)

        def idx_body(j, off):
            p16 = j * 16 + iota
            x = x_v[pl.ds(j * 16, 16)]
            y = y_v[pl.ds(j * 16, 16)]
            z = z_v[pl.ds(j * 16, 16)]
            ix = jnp.clip((x * float(NB) + float(NB // 2)).astype(jnp.int32), 0, NB - 1)
            iy = jnp.clip((y * float(NB) + float(NB // 2)).astype(jnp.int32), 0, NB - 1)
            iz = jnp.clip((z * float(NB) + float(NB // 2)).astype(jnp.int32), 0, NB - 1)
            q0 = (ix * NB + iy) * 32 + (iz >> 4)
            cond = ((jnp.abs(x) < 0.5) & (jnp.abs(y) < 0.5) & (jnp.abs(z) < 0.5))
            plsc.store_compressed(q_v.at[pl.ds(off, 16)], q0, mask=cond)
            plsc.store_compressed(col_v.at[pl.ds(off, 16)], iz & 15, mask=cond)
            plsc.store_compressed(pos_v.at[pl.ds(off, 16)], p16, mask=cond)
            n = plsc.all_reduce_population_count(cond)
            return off + n[0]

        ncomp = lax.fori_loop(0, BLK // 16, idx_body, jnp.int32(0))
        nent = ncomp * 4

        def bld_body(t, c):
            e = t * 16 + iota
            cp = e >> 2
            q = plsc.load_gather(q_v, [cp])
            idx_v[pl.ds(t * 16, 16)] = q + (iota & 3) * 8
            return c

        lax.fori_loop(0, (nent + 15) >> 4, bld_body, 0)

        def init_rgb(t, c):
            rgb_v[pl.ds(t * 16, 16)] = half
            return c
        lax.fori_loop(0, BLK * 3 // 16, init_rgb, 0)

        def init_dens(t, c):
            dens_v[pl.ds(t * 16, 16)] = zerov
            return c
        lax.fori_loop(0, BLK // 16, init_dens, 0)

        for w in range(NWAVE):
            @pl.when(w * CHUNK < nent)
            def _():
                pltpu.async_copy(
                    table_hbm.at[idx_v.at[pl.ds(w * CHUNK, CHUNK)]],
                    rows_v, sem).wait()

                def val_body(t, c):
                    le = t * 16 + iota
                    e = w * CHUNK + le
                    cp = e >> 2
                    ch = iota & 3
                    valid = le < (nent - w * CHUNK)
                    kcol = plsc.load_gather(col_v, [cp]) & 15
                    vals = plsc.load_gather(rows_v, [le, kcol])
                    pos = plsc.load_gather(pos_v, [cp])
                    sig = 1.0 / (1.0 + jnp.exp(-vals))
                    rel = jnp.maximum(vals, 0.0)
                    out = jnp.where(ch < 3, sig, rel)
                    plsc.store_scatter(rgb_v, [pos * 3 + ch], out,
                                       mask=(ch < 3) & valid)
                    plsc.store_scatter(dens_v, [pos], out,
                                       mask=(ch == 3) & valid)
                    return c

                nv = jnp.minimum(nent - w * CHUNK + 15, CHUNK + 15) >> 4
                lax.fori_loop(0, nv, val_body, 0)

        pltpu.sync_copy(rgb_v, rgb_hbm.at[pl.ds(base * 3, BLK * 3)])
        pltpu.sync_copy(dens_v, dens_hbm.at[pl.ds(base, BLK)])
        return carry

    lax.fori_loop(0, NBLK, block_body, 0)


def kernel(xyz, d, voxels):
    del d
    table = voxels.transpose(0, 1, 3, 2).reshape(RT, 16)
    x = xyz[:, 0]
    y = xyz[:, 1]
    z = xyz[:, 2]
    rgb_flat, dens_flat = _voxel_fwd(table, x, y, z)
    return rgb_flat.reshape(N_PTS, 3), dens_flat.reshape(N_PTS, 1)


# speedup vs baseline: 1.1192x; 1.1192x over previous
import functools

import jax
import jax.numpy as jnp
from jax import lax
from jax.experimental import pallas as pl
from jax.experimental.pallas import tpu as pltpu
from jax.experimental.pallas import tpu_sc as plsc

N_PTS = 1048576
NB = 128
RT = NB * NB * NB * 4 // 16

NC = 2
NS = 16
NW = NC * NS

BLK = 1024
PER_W = N_PTS // NW
NBLK = PER_W // BLK
CHUNK = 512
NCHUNK = BLK * 4 // CHUNK

_mesh = plsc.VectorSubcoreMesh(core_axis_name="c", subcore_axis_name="s")


@functools.partial(
    pl.kernel,
    mesh=_mesh,
    compiler_params=pltpu.CompilerParams(
        needs_layout_passes=False, use_tc_tiling_on_sc=False),
    out_type=[
        jax.ShapeDtypeStruct((N_PTS * 3,), jnp.float32),
        jax.ShapeDtypeStruct((N_PTS,), jnp.float32),
    ],
    scratch_types=[
        pltpu.VMEM((BLK,), jnp.float32),
        pltpu.VMEM((BLK,), jnp.float32),
        pltpu.VMEM((BLK,), jnp.float32),
        pltpu.VMEM((BLK + 16,), jnp.int32),
        pltpu.VMEM((BLK + 16,), jnp.int32),
        pltpu.VMEM((BLK + 16,), jnp.int32),
        pltpu.VMEM((BLK * 4,), jnp.int32),
        pltpu.VMEM((BLK * 4, 16), jnp.float32),
        pltpu.VMEM((BLK * 3 + 16,), jnp.float32),
        pltpu.VMEM((BLK + 16,), jnp.float32),
        pltpu.SemaphoreType.DMA,
    ],
)
def _voxel_fwd(table_hbm, x_hbm, y_hbm, z_hbm, half_hbm, zero_hbm,
               rgb_hbm, dens_hbm,
               x_v, y_v, z_v, q_v, col_v, pos_v, idx_v, rows_v,
               rgb_v, dens_v, sem):
    wid = lax.axis_index("s") * NC + lax.axis_index("c")
    iota = lax.iota(jnp.int32, 16)
    zeroi = jnp.zeros((16,), jnp.int32)

    def init_idx(t, c):
        idx_v[pl.ds(t * 16, 16)] = zeroi
        return c
    lax.fori_loop(0, BLK * 4 // 16, init_idx, 0)

    def block_body(b, carry):
        base = (wid * NBLK + b) * BLK
        pltpu.sync_copy(x_hbm.at[pl.ds(base, BLK)], x_v)
        pltpu.sync_copy(y_hbm.at[pl.ds(base, BLK)], y_v)
        pltpu.sync_copy(z_hbm.at[pl.ds(base, BLK)], z_v)
        pltpu.sync_copy(half_hbm, rgb_v)
        pltpu.sync_copy(zero_hbm, dens_v)

        def one_idx(j, off):
            p16 = j * 16 + iota
            x = x_v[pl.ds(j * 16, 16)]
            y = y_v[pl.ds(j * 16, 16)]
            z = z_v[pl.ds(j * 16, 16)]
            ix = jnp.clip((x * float(NB) + float(NB // 2)).astype(jnp.int32), 0, NB - 1)
            iy = jnp.clip((y * float(NB) + float(NB // 2)).astype(jnp.int32), 0, NB - 1)
            iz = jnp.clip((z * float(NB) + float(NB // 2)).astype(jnp.int32), 0, NB - 1)
            q0 = (ix * NB + iy) * 32 + (iz >> 4)
            cond = ((jnp.abs(x) < 0.5) & (jnp.abs(y) < 0.5) & (jnp.abs(z) < 0.5))
            plsc.store_compressed(q_v.at[pl.ds(off, 16)], q0, mask=cond)
            plsc.store_compressed(col_v.at[pl.ds(off, 16)], iz & 15, mask=cond)
            plsc.store_compressed(pos_v.at[pl.ds(off, 16)], p16, mask=cond)
            n = plsc.all_reduce_population_count(cond)
            return off + n[0]

        def idx_body(jj, off):
            off = one_idx(jj * 2, off)
            return one_idx(jj * 2 + 1, off)

        ncomp = lax.fori_loop(0, BLK // 32, idx_body, jnp.int32(0))

        q_v[pl.ds(ncomp, 16)] = zeroi
        col_v[pl.ds(ncomp, 16)] = zeroi
        pos_v[pl.ds(ncomp, 16)] = jnp.full((16,), BLK, jnp.int32)
        npad = (ncomp + 7) & ~7
        nent = npad * 4

        def one_bld(t):
            e = t * 16 + iota
            cp = e >> 2
            q = plsc.load_gather(q_v, [cp])
            idx_v[pl.ds(t * 16, 16)] = q + (iota & 3) * 8

        def bld_body(tt, c):
            one_bld(tt * 2)
            one_bld(tt * 2 + 1)
            return c

        lax.fori_loop(0, nent >> 5, bld_body, 0)

        for i in range(NCHUNK):
            @pl.when(i * CHUNK < nent)
            def _():
                pltpu.async_copy(
                    table_hbm.at[idx_v.at[pl.ds(i * CHUNK, CHUNK)]],
                    rows_v.at[pl.ds(i * CHUNK, CHUNK), :], sem).wait()

        def one_val(t):
            e = t * 16 + iota
            cp = e >> 2
            ch = iota & 3
            kcol = plsc.load_gather(col_v, [cp])
            vals = plsc.load_gather(rows_v, [e, kcol])
            pos = plsc.load_gather(pos_v, [cp])
            sig = 1.0 / (1.0 + jnp.exp(-vals))
            rel = jnp.maximum(vals, 0.0)
            out = jnp.where(ch < 3, sig, rel)
            plsc.store_scatter(rgb_v, [pos * 3 + ch], out, mask=ch < 3)
            plsc.store_scatter(dens_v, [pos], out, mask=ch == 3)

        def val_body(tt, c):
            one_val(tt * 2)
            one_val(tt * 2 + 1)
            return c

        lax.fori_loop(0, nent >> 5, val_body, 0)

        pltpu.sync_copy(rgb_v.at[pl.ds(0, BLK * 3)], rgb_hbm.at[pl.ds(base * 3, BLK * 3)])
        pltpu.sync_copy(dens_v.at[pl.ds(0, BLK)], dens_hbm.at[pl.ds(base, BLK)])
        return carry

    lax.fori_loop(0, NBLK, block_body, 0)


def kernel(xyz, d, voxels):
    del d
    table = voxels.transpose(0, 1, 3, 2).reshape(RT, 16)
    x = xyz[:, 0]
    y = xyz[:, 1]
    z = xyz[:, 2]
    half_c = jnp.full((BLK * 3 + 16,), 0.5, jnp.float32)
    zero_c = jnp.zeros((BLK + 16,), jnp.float32)
    rgb_flat, dens_flat = _voxel_fwd(table, x, y, z, half_c, zero_c)
    return rgb_flat.reshape(N_PTS, 3), dens_flat.reshape(N_PTS, 1)
